# baseline (speedup 1.0000x reference)
import functools

import jax
import jax.numpy as jnp
from jax import lax
from jax.experimental import pallas as pl
from jax.experimental.pallas import tpu as pltpu
from jax.experimental.pallas import tpu_sc as plsc

_NC = 2
_NS = 16
_LANES = 16
_CH = 8


def kernel(x, pos_emb):
    B, S, D = x.shape
    BS = B * S
    NW = _NC * _NS
    rows_per_w = BS // NW
    wpb = NW // B
    srows = S // wpb
    xf = x.reshape(BS, D)
    vecs_per_row = D // _LANES

    mesh = plsc.VectorSubcoreMesh(core_axis_name="c", subcore_axis_name="s")

    @functools.partial(
        pl.kernel,
        out_type=jax.ShapeDtypeStruct((BS, D), jnp.float32),
        mesh=mesh,
        scratch_types=[
            pltpu.VMEM((_CH, D), jnp.float32),
            pltpu.VMEM((_CH, D), jnp.float32),
        ],
    )
    def sc_add(x_hbm, pe_hbm, out_hbm, xbuf, pebuf):
        wid = lax.axis_index("s") * _NC + lax.axis_index("c")
        b = wid // wpb
        slot = wid % wpb
        xbase = b * S + slot * srows
        pbase = slot * srows

        def chunk(i, carry):
            xo = xbase + i * _CH
            po = pbase + i * _CH
            pltpu.sync_copy(x_hbm.at[pl.ds(xo, _CH)], xbuf)
            pltpu.sync_copy(pe_hbm.at[pl.ds(po, _CH)], pebuf)

            def col(j, c2):
                r = j // vecs_per_row
                cb = (j % vecs_per_row) * _LANES
                v = pebuf[r, pl.ds(cb, _LANES)]
                plsc.addupdate(xbuf.at[r, pl.ds(cb, _LANES)], v)
                return c2

            lax.fori_loop(0, _CH * vecs_per_row, col, 0)
            pltpu.sync_copy(xbuf, out_hbm.at[pl.ds(xo, _CH)])
            return carry

        lax.fori_loop(0, srows // _CH, chunk, 0)

    return sc_add(xf, pos_emb).reshape(B, S, D)


# speedup vs baseline: 4.9051x; 4.9051x over previous
import jax
import jax.numpy as jnp
from jax.experimental import pallas as pl
from jax.experimental.pallas import tpu as pltpu


_SBLK = 512


def _add_tile(x_ref, pe_ref, o_ref):
    o_ref[...] = x_ref[...] + pe_ref[...]


def kernel(x, pos_emb):
    B, S, D = x.shape
    sblk = _SBLK if S % _SBLK == 0 else S
    grid = (S // sblk, B)
    return pl.pallas_call(
        _add_tile,
        grid=grid,
        in_specs=[
            pl.BlockSpec((1, sblk, D), lambda s, b: (b, s, 0)),
            pl.BlockSpec((sblk, D), lambda s, b: (s, 0)),
        ],
        out_specs=pl.BlockSpec((1, sblk, D), lambda s, b: (b, s, 0)),
        out_shape=jax.ShapeDtypeStruct((B, S, D), x.dtype),
        compiler_params=pltpu.CompilerParams(
            dimension_semantics=("parallel", "parallel"),
            vmem_limit_bytes=60 * 1024 * 1024,
        ),
    )(x, pos_emb)
